# baseline (speedup 1.0000x reference)
import functools

import jax
import jax.numpy as jnp
from jax import lax
from jax.experimental import pallas as pl
from jax.experimental.pallas import tpu as pltpu
from jax.experimental.pallas import tpu_sc as plsc

NC = 2
NS = 16
NW = NC * NS


def _make_deg_kernel(n, ch, b):
    mesh = plsc.VectorSubcoreMesh(core_axis_name="c", subcore_axis_name="s")
    rows_per_sub = n // NS
    f = 16

    @functools.partial(
        pl.kernel,
        out_type=jax.ShapeDtypeStruct((NC, n, f), jnp.float32),
        mesh=mesh,
        scratch_types=[
            pltpu.VMEM((ch, b), jnp.int32),
            pltpu.VMEM((b, f), jnp.float32),
            pltpu.VMEM_SHARED((n, f), jnp.float32),
        ],
        compiler_params=pltpu.CompilerParams(use_tc_tiling_on_sc=False),
    )
    def deg_kernel(dst_hbm, zeros_hbm, ones_hbm, out_hbm, idx_d, ones_v, acc):
        c = lax.axis_index("c")
        s = lax.axis_index("s")
        w = s * NC + c
        pltpu.sync_copy(ones_hbm, ones_v)
        pltpu.sync_copy(zeros_hbm.at[pl.ds(s * rows_per_sub, rows_per_sub)],
                        acc.at[pl.ds(s * rows_per_sub, rows_per_sub)])
        plsc.subcore_barrier()
        pltpu.sync_copy(dst_hbm.at[w], idx_d)

        @pl.loop(0, ch)
        def _(j):
            pltpu.sync_copy(ones_v, acc.at[idx_d.at[j]], add=True)

        plsc.subcore_barrier()
        pltpu.sync_copy(acc.at[pl.ds(s * rows_per_sub, rows_per_sub)],
                        out_hbm.at[c, pl.ds(s * rows_per_sub, rows_per_sub)])

    return deg_kernel


def _make_segsum_kernel(n, f, ch, b, nb=5):
    mesh = plsc.VectorSubcoreMesh(core_axis_name="c", subcore_axis_name="s")
    rows_per_sub = n // NS
    grps = ch // nb

    @functools.partial(
        pl.kernel,
        out_type=jax.ShapeDtypeStruct((NC, n, f), jnp.float32),
        mesh=mesh,
        scratch_types=[
            pltpu.VMEM((ch, b), jnp.int32),
            pltpu.VMEM((ch, b), jnp.int32),
            pltpu.VMEM((nb, b, f), jnp.float32),
            pltpu.VMEM_SHARED((n, f), jnp.float32),
            pltpu.VMEM_SHARED((n, f), jnp.float32),
            pltpu.SemaphoreType.DMA,
            pltpu.SemaphoreType.DMA,
        ],
        compiler_params=pltpu.CompilerParams(use_tc_tiling_on_sc=False),
    )
    def seg_kernel(src_hbm, dst_hbm, table_hbm, zeros_hbm, out_hbm,
                   idx_s, idx_d, rows, acc, stab, gsem, ssem):
        c = lax.axis_index("c")
        s = lax.axis_index("s")
        w = s * NC + c
        pltpu.sync_copy(zeros_hbm.at[pl.ds(s * rows_per_sub, rows_per_sub)],
                        acc.at[pl.ds(s * rows_per_sub, rows_per_sub)])
        pltpu.sync_copy(table_hbm.at[pl.ds(s * rows_per_sub, rows_per_sub)],
                        stab.at[pl.ds(s * rows_per_sub, rows_per_sub)])
        plsc.subcore_barrier()
        pltpu.sync_copy(src_hbm.at[w], idx_s)
        pltpu.sync_copy(dst_hbm.at[w], idx_d)

        for slot in range(nb):
            pltpu.async_copy(stab.at[idx_s.at[slot]], rows.at[slot], gsem)

        @pl.loop(0, grps)
        def _(g):
            for slot in range(nb):
                j = g * nb + slot
                pltpu.make_async_copy(
                    stab.at[idx_s.at[j]], rows.at[slot], gsem).wait()
                sd = pltpu.async_copy(
                    rows.at[slot], acc.at[idx_d.at[j]], ssem, add=True)
                sd.wait()

                @pl.when(g < grps - 1)
                def _():
                    pltpu.async_copy(
                        stab.at[idx_s.at[j + nb]], rows.at[slot], gsem)

        plsc.subcore_barrier()
        pltpu.sync_copy(acc.at[pl.ds(s * rows_per_sub, rows_per_sub)],
                        out_hbm.at[c, pl.ds(s * rows_per_sub, rows_per_sub)])

    return seg_kernel


def _make_scale1_body(n, npad, f):
    def body(x_ref, w1_ref, degp_ref, g1_ref, dis_ref):
        deg = degp_ref[0, :n, 0:1] + degp_ref[1, :n, 0:1] + 1.0
        dis_col = lax.rsqrt(deg)
        h1 = jnp.dot(x_ref[...], w1_ref[...],
                     preferred_element_type=jnp.float32)
        g1_ref[:n, :] = dis_col * h1
        g1_ref[n:, :] = jnp.zeros((npad - n, f), jnp.float32)
        dis_ref[...] = dis_col
    return body


def _make_layer2_body(n, npad, f, h, cdim):
    def body(s1p_ref, g1_ref, dis_ref, b1_ref, w2_ref, g2_ref):
        hmat = jnp.maximum(
            dis_ref[...] * (s1p_ref[0, :n, :] + s1p_ref[1, :n, :]
                            + g1_ref[:n, :]) + b1_ref[...], 0.0)
        prod = jnp.dot(hmat[:, :h], w2_ref[...],
                       preferred_element_type=jnp.float32)
        g2_ref[:n, :] = dis_ref[...] * jnp.concatenate(
            [prod, jnp.zeros((n, f - cdim), jnp.float32)], axis=1)
        g2_ref[n:, :] = jnp.zeros((npad - n, f), jnp.float32)
    return body


def _make_final_body(n, f, cdim):
    def body(s2p_ref, g2_ref, dis_ref, b2_ref, out_ref):
        out_ref[...] = (dis_ref[...]
                        * (s2p_ref[0, :n, :cdim] + s2p_ref[1, :n, :cdim]
                           + g2_ref[:n, :cdim]) + b2_ref[...])
    return body


@jax.jit
def kernel(x, edge_index, W1, b1, W2, b2):
    n, d = x.shape
    h = W1.shape[1]
    cdim = W2.shape[1]
    e = edge_index.shape[1]
    f = 16

    npad = ((n + NS * 8 - 1) // (NS * 8)) * NS * 8
    b_chunk = 128
    nb = 5
    per_w = e // NW
    ch = (per_w + b_chunk - 1) // b_chunk
    ch = ((ch + nb - 1) // nb) * nb
    pw_pad = ch * b_chunk

    src_pw = edge_index[0].reshape(NW, per_w)
    dst_pw = edge_index[1].reshape(NW, per_w)
    pad = pw_pad - per_w
    if pad:
        src_pw = jnp.pad(src_pw, ((0, 0), (0, pad)))
        dst_pw = jnp.pad(dst_pw, ((0, 0), (0, pad)), constant_values=n)
    src3 = src_pw.reshape(NW, ch, b_chunk)
    dst3 = dst_pw.reshape(NW, ch, b_chunk)

    ones_b = jnp.ones((b_chunk, f), jnp.float32)
    zeros_nf = jnp.zeros((npad, f), jnp.float32)

    deg_p3 = _make_deg_kernel(npad, ch, b_chunk)(dst3, zeros_nf, ones_b)

    g1, dis = pl.pallas_call(
        _make_scale1_body(n, npad, f),
        out_shape=(jax.ShapeDtypeStruct((npad, f), jnp.float32),
                   jax.ShapeDtypeStruct((n, 1), jnp.float32)),
    )(x, W1, deg_p3)

    s1_pp = _make_segsum_kernel(npad, f, ch, b_chunk)(src3, dst3, g1, zeros_nf)

    g2 = pl.pallas_call(
        _make_layer2_body(n, npad, f, h, cdim),
        out_shape=jax.ShapeDtypeStruct((npad, f), jnp.float32),
    )(s1_pp, g1, dis, b1.reshape(1, h), W2)

    s2_pp = _make_segsum_kernel(npad, f, ch, b_chunk)(src3, dst3, g2, zeros_nf)

    out = pl.pallas_call(
        _make_final_body(n, f, cdim),
        out_shape=jax.ShapeDtypeStruct((n, cdim), jnp.float32),
    )(s2_pp, g2, dis, b2.reshape(1, cdim))

    return out


# speedup vs baseline: 1.0163x; 1.0163x over previous
import functools

import jax
import jax.numpy as jnp
from jax import lax
from jax.experimental import pallas as pl
from jax.experimental.pallas import tpu as pltpu
from jax.experimental.pallas import tpu_sc as plsc

NC = 2
NS = 16
NW = NC * NS


def _make_deg_kernel(n, ch, b):
    mesh = plsc.VectorSubcoreMesh(core_axis_name="c", subcore_axis_name="s")
    rows_per_sub = n // NS
    f = 16

    @functools.partial(
        pl.kernel,
        out_type=jax.ShapeDtypeStruct((NC, n, f), jnp.float32),
        mesh=mesh,
        scratch_types=[
            pltpu.VMEM((ch, b), jnp.int32),
            pltpu.VMEM((b, f), jnp.float32),
            pltpu.VMEM_SHARED((n, f), jnp.float32),
        ],
        compiler_params=pltpu.CompilerParams(use_tc_tiling_on_sc=False),
    )
    def deg_kernel(dst_hbm, zeros_hbm, ones_hbm, out_hbm, idx_d, ones_v, acc):
        c = lax.axis_index("c")
        s = lax.axis_index("s")
        w = s * NC + c
        pltpu.sync_copy(ones_hbm, ones_v)
        pltpu.sync_copy(zeros_hbm.at[pl.ds(s * rows_per_sub, rows_per_sub)],
                        acc.at[pl.ds(s * rows_per_sub, rows_per_sub)])
        plsc.subcore_barrier()
        pltpu.sync_copy(dst_hbm.at[w], idx_d)

        @pl.loop(0, ch)
        def _(j):
            pltpu.sync_copy(ones_v, acc.at[idx_d.at[j]], add=True)

        plsc.subcore_barrier()
        pltpu.sync_copy(acc.at[pl.ds(s * rows_per_sub, rows_per_sub)],
                        out_hbm.at[c, pl.ds(s * rows_per_sub, rows_per_sub)])

    return deg_kernel


def _make_segsum_kernel(n, f, ch, b, nb=5):
    mesh = plsc.VectorSubcoreMesh(core_axis_name="c", subcore_axis_name="s")
    rows_per_sub = n // NS
    grps = ch // nb

    @functools.partial(
        pl.kernel,
        out_type=jax.ShapeDtypeStruct((NC, n, f), jnp.float32),
        mesh=mesh,
        scratch_types=[
            pltpu.VMEM((ch, b), jnp.int32),
            pltpu.VMEM((ch, b), jnp.int32),
            pltpu.VMEM((nb, b, f), jnp.float32),
            pltpu.VMEM_SHARED((n, f), jnp.float32),
            pltpu.VMEM_SHARED((n, f), jnp.float32),
            pltpu.SemaphoreType.DMA,
            pltpu.SemaphoreType.DMA,
        ],
        compiler_params=pltpu.CompilerParams(use_tc_tiling_on_sc=False),
    )
    def seg_kernel(src_hbm, dst_hbm, table_hbm, zeros_hbm, out_hbm,
                   idx_s, idx_d, rows, acc, stab, gsem, ssem):
        c = lax.axis_index("c")
        s = lax.axis_index("s")
        w = s * NC + c
        pltpu.sync_copy(zeros_hbm.at[pl.ds(s * rows_per_sub, rows_per_sub)],
                        acc.at[pl.ds(s * rows_per_sub, rows_per_sub)])
        pltpu.sync_copy(table_hbm.at[pl.ds(s * rows_per_sub, rows_per_sub)],
                        stab.at[pl.ds(s * rows_per_sub, rows_per_sub)])
        plsc.subcore_barrier()
        pltpu.sync_copy(src_hbm.at[w], idx_s)
        pltpu.sync_copy(dst_hbm.at[w], idx_d)

        for slot in range(nb):
            pltpu.async_copy(stab.at[idx_s.at[slot]], rows.at[slot], gsem)

        @pl.loop(0, grps)
        def _(g):
            for slot in range(nb):
                j = g * nb + slot
                pltpu.make_async_copy(
                    stab.at[idx_s.at[j]], rows.at[slot], gsem).wait()
                sd = pltpu.async_copy(
                    rows.at[slot], acc.at[idx_d.at[j]], ssem, add=True)
                sd.wait()

                @pl.when(g < grps - 1)
                def _():
                    pltpu.async_copy(
                        stab.at[idx_s.at[j + nb]], rows.at[slot], gsem)

        plsc.subcore_barrier()
        pltpu.sync_copy(acc.at[pl.ds(s * rows_per_sub, rows_per_sub)],
                        out_hbm.at[c, pl.ds(s * rows_per_sub, rows_per_sub)])

    return seg_kernel


def _make_scale1_body(n, npad, f):
    def body(x_ref, w1_ref, degp_ref, g1_ref, dis_ref):
        deg = degp_ref[0, :n, 0:1] + degp_ref[1, :n, 0:1] + 1.0
        dis_col = lax.rsqrt(deg)
        h1 = jnp.dot(x_ref[...], w1_ref[...],
                     preferred_element_type=jnp.float32)
        g1_ref[:n, :] = dis_col * h1
        g1_ref[n:, :] = jnp.zeros((npad - n, f), jnp.float32)
        dis_ref[...] = dis_col
    return body


def _make_layer2_body(n, npad, f, h, cdim):
    def body(s1p_ref, g1_ref, dis_ref, b1_ref, w2_ref, g2_ref):
        hmat = jnp.maximum(
            dis_ref[...] * (s1p_ref[0, :n, :] + s1p_ref[1, :n, :]
                            + g1_ref[:n, :]) + b1_ref[...], 0.0)
        prod = jnp.dot(hmat[:, :h], w2_ref[...],
                       preferred_element_type=jnp.float32)
        g2_ref[:n, :] = dis_ref[...] * jnp.concatenate(
            [prod, jnp.zeros((n, f - cdim), jnp.float32)], axis=1)
        g2_ref[n:, :] = jnp.zeros((npad - n, f), jnp.float32)
    return body


def _make_final_body(n, f, cdim):
    def body(s2p_ref, g2_ref, dis_ref, b2_ref, out_ref):
        out_ref[...] = (dis_ref[...]
                        * (s2p_ref[0, :n, :cdim] + s2p_ref[1, :n, :cdim]
                           + g2_ref[:n, :cdim]) + b2_ref[...])
    return body


@jax.jit
def kernel(x, edge_index, W1, b1, W2, b2):
    n, d = x.shape
    h = W1.shape[1]
    cdim = W2.shape[1]
    e = edge_index.shape[1]
    f = 16

    npad = ((n + NS * 8 - 1) // (NS * 8)) * NS * 8
    b_chunk = 80
    nb = 5
    per_w = e // NW
    ch = (per_w + b_chunk - 1) // b_chunk
    ch = ((ch + nb - 1) // nb) * nb
    pw_pad = ch * b_chunk

    src_pw = edge_index[0].reshape(NW, per_w)
    dst_pw = edge_index[1].reshape(NW, per_w)
    pad = pw_pad - per_w
    if pad:
        src_pw = jnp.pad(src_pw, ((0, 0), (0, pad)))
        dst_pw = jnp.pad(dst_pw, ((0, 0), (0, pad)), constant_values=n)
    src3 = src_pw.reshape(NW, ch, b_chunk)
    dst3 = dst_pw.reshape(NW, ch, b_chunk)

    ones_b = jnp.ones((b_chunk, f), jnp.float32)
    zeros_nf = jnp.zeros((npad, f), jnp.float32)

    deg_p3 = _make_deg_kernel(npad, ch, b_chunk)(dst3, zeros_nf, ones_b)

    g1, dis = pl.pallas_call(
        _make_scale1_body(n, npad, f),
        out_shape=(jax.ShapeDtypeStruct((npad, f), jnp.float32),
                   jax.ShapeDtypeStruct((n, 1), jnp.float32)),
    )(x, W1, deg_p3)

    s1_pp = _make_segsum_kernel(npad, f, ch, b_chunk)(src3, dst3, g1, zeros_nf)

    g2 = pl.pallas_call(
        _make_layer2_body(n, npad, f, h, cdim),
        out_shape=jax.ShapeDtypeStruct((npad, f), jnp.float32),
    )(s1_pp, g1, dis, b1.reshape(1, h), W2)

    s2_pp = _make_segsum_kernel(npad, f, ch, b_chunk)(src3, dst3, g2, zeros_nf)

    out = pl.pallas_call(
        _make_final_body(n, f, cdim),
        out_shape=jax.ShapeDtypeStruct((n, cdim), jnp.float32),
    )(s2_pp, g2, dis, b2.reshape(1, cdim))

    return out
